# initial kernel scaffold (unmeasured)
import jax
import jax.numpy as jnp
from jax import lax
from jax.experimental import pallas as pl
from jax.experimental.pallas import tpu as pltpu


def kernel(
    x,
):
    def body(*refs):
        pass

    out_shape = jax.ShapeDtypeStruct(..., jnp.float32)
    return pl.pallas_call(body, out_shape=out_shape)(...)


# baseline (device time: 1590843 ns/iter reference)
import jax
import jax.numpy as jnp
from jax import lax
from jax.experimental import pallas as pl
from jax.experimental.pallas import tpu as pltpu


CHUNK = 2048


def kernel(x):
    m, n = x.shape
    grid = m // CHUNK

    def body(x_ref, out_ref, recv_ref, send_sem, recv_sem, ready_sem):
        my_x = lax.axis_index("x")
        my_y = lax.axis_index("y")
        my_z = lax.axis_index("z")
        partner = (1 - my_x, my_y, my_z)

        rdma = pltpu.make_async_remote_copy(
            src_ref=x_ref,
            dst_ref=recv_ref,
            send_sem=send_sem,
            recv_sem=recv_sem,
            device_id=partner,
            device_id_type=pl.DeviceIdType.MESH,
        )
        rdma.start()
        rdma.wait()

        out_ref[...] = x_ref[...] + recv_ref[...]

        pl.semaphore_signal(
            ready_sem,
            inc=1,
            device_id=partner,
            device_id_type=pl.DeviceIdType.MESH,
        )
        pl.semaphore_wait(ready_sem, 1)

    return pl.pallas_call(
        body,
        grid=(grid,),
        out_shape=jax.ShapeDtypeStruct((m, n), x.dtype),
        in_specs=[pl.BlockSpec((CHUNK, n), lambda i: (i, 0))],
        out_specs=pl.BlockSpec((CHUNK, n), lambda i: (i, 0)),
        scratch_shapes=[
            pltpu.VMEM((CHUNK, n), x.dtype),
            pltpu.SemaphoreType.DMA,
            pltpu.SemaphoreType.DMA,
            pltpu.SemaphoreType.REGULAR,
        ],
        compiler_params=pltpu.CompilerParams(
            dimension_semantics=("arbitrary",),
            vmem_limit_bytes=100 * 1024 * 1024,
        ),
    )(x)


# device time: 767233 ns/iter; 2.0735x vs baseline; 2.0735x over previous
import jax
import jax.numpy as jnp
from jax import lax
from jax.experimental import pallas as pl
from jax.experimental.pallas import tpu as pltpu


M = 32768
N = 1024
Q = M // 4
CH = 2048
NC = Q // CH
N_SEMS = 16


def _comm_body(x_ref, rem_ref, recv_sems, send_sems):
    my_x = lax.axis_index("x")
    my_y = lax.axis_index("y")
    my_z = lax.axis_index("z")
    xp = (1 - my_x, my_y, my_z)
    yp = (my_x, 1 - my_y, my_z)
    zp = (my_x, my_y, 1 - my_z)

    q = 2 * my_y + my_z
    qy = 2 * (1 - my_y) + my_z
    qz = 2 * my_y + (1 - my_z)
    d = 2 * (1 - my_y) + (1 - my_z)

    def mk(src_base, row, sem_idx, dev):
        return pltpu.make_async_remote_copy(
            src_ref=src_base.at[pl.ds(row, CH), :],
            dst_ref=rem_ref.at[pl.ds(row, CH), :],
            send_sem=send_sems.at[sem_idx],
            recv_sem=recv_sems.at[sem_idx],
            device_id=dev,
            device_id_type=pl.DeviceIdType.MESH,
        )

    A = []
    for c in range(NC):
        r = mk(x_ref, q * Q + c * CH, c, xp)
        r.start()
        A.append(r)

    By, Bz = [], []
    for c in range(NC):
        A[c].wait_recv()
        ry = mk(rem_ref, q * Q + c * CH, 4 + c, yp)
        ry.start()
        By.append(ry)
        rz = mk(rem_ref, q * Q + c * CH, 8 + c, zp)
        rz.start()
        Bz.append(rz)

    Cy, Cz = [], []
    for c in range(NC):
        Bz[c].wait_recv()
        if c < 2:
            r = mk(rem_ref, qz * Q + c * CH, 12 + c, yp)
            r.start()
            Cy.append(r)
        By[c].wait_recv()
        if c >= 2:
            r = mk(rem_ref, qy * Q + c * CH, 14 + (c - 2), zp)
            r.start()
            Cz.append(r)

    for c in range(2):
        Cy[c].wait_recv()
        Cz[c].wait_recv()

    for r in A + By + Bz + Cy + Cz:
        r.wait_send()


def _add_body(x_ref, rem_ref, out_ref):
    out_ref[...] = x_ref[...] + rem_ref[...]


def kernel(x):
    m, n = x.shape
    assert (m, n) == (M, N)

    rem = pl.pallas_call(
        _comm_body,
        out_shape=jax.ShapeDtypeStruct((m, n), x.dtype),
        in_specs=[pl.BlockSpec(memory_space=pltpu.MemorySpace.HBM)],
        out_specs=pl.BlockSpec(memory_space=pltpu.MemorySpace.HBM),
        scratch_shapes=[
            pltpu.SemaphoreType.DMA((N_SEMS,)),
            pltpu.SemaphoreType.DMA((N_SEMS,)),
        ],
    )(x)

    return pl.pallas_call(
        _add_body,
        grid=(m // CH,),
        out_shape=jax.ShapeDtypeStruct((m, n), x.dtype),
        in_specs=[
            pl.BlockSpec((CH, n), lambda i: (i, 0)),
            pl.BlockSpec((CH, n), lambda i: (i, 0)),
        ],
        out_specs=pl.BlockSpec((CH, n), lambda i: (i, 0)),
        compiler_params=pltpu.CompilerParams(
            dimension_semantics=("arbitrary",),
            vmem_limit_bytes=100 * 1024 * 1024,
        ),
    )(x, rem)


# device time: 749737 ns/iter; 2.1219x vs baseline; 1.0233x over previous
import jax
import jax.numpy as jnp
from jax import lax
from jax.experimental import pallas as pl
from jax.experimental.pallas import tpu as pltpu


M = 32768
N = 1024
Q = M // 4
CH = 2048
NC = Q // CH
N_SEMS = 16


def _body(x_ref, out_ref, rem_ref, recv_sems, send_sems, lsems,
          vx_ref, vr_ref, vo_ref):
    my_x = lax.axis_index("x")
    my_y = lax.axis_index("y")
    my_z = lax.axis_index("z")
    xp = (1 - my_x, my_y, my_z)
    yp = (my_x, 1 - my_y, my_z)
    zp = (my_x, my_y, 1 - my_z)

    q = 2 * my_y + my_z
    qy = 2 * (1 - my_y) + my_z
    qz = 2 * my_y + (1 - my_z)
    d = 2 * (1 - my_y) + (1 - my_z)

    def mk(src_base, row, sem_idx, dev):
        return pltpu.make_async_remote_copy(
            src_ref=src_base.at[pl.ds(row, CH), :],
            dst_ref=rem_ref.at[pl.ds(row, CH), :],
            send_sem=send_sems.at[sem_idx],
            recv_sem=recv_sems.at[sem_idx],
            device_id=dev,
            device_id_type=pl.DeviceIdType.MESH,
        )

    def add_chunk(row):
        cx = pltpu.make_async_copy(
            x_ref.at[pl.ds(row, CH), :], vx_ref, lsems.at[0])
        cr = pltpu.make_async_copy(
            rem_ref.at[pl.ds(row, CH), :], vr_ref, lsems.at[1])
        cx.start()
        cr.start()
        cx.wait()
        cr.wait()
        vo_ref[...] = vx_ref[...] + vr_ref[...]
        co = pltpu.make_async_copy(
            vo_ref, out_ref.at[pl.ds(row, CH), :], lsems.at[2])
        co.start()
        co.wait()

    A = []
    for c in range(NC):
        r = mk(x_ref, q * Q + c * CH, c, xp)
        r.start()
        A.append(r)

    By, Bz = [], []
    for c in range(NC):
        A[c].wait_recv()
        ry = mk(rem_ref, q * Q + c * CH, 4 + c, yp)
        ry.start()
        By.append(ry)
        rz = mk(rem_ref, q * Q + c * CH, 8 + c, zp)
        rz.start()
        Bz.append(rz)
        add_chunk(q * Q + c * CH)

    Cy, Cz = [], []
    for c in range(NC):
        Bz[c].wait_recv()
        if c < 2:
            r = mk(rem_ref, qz * Q + c * CH, 12 + c, yp)
            r.start()
            Cy.append(r)
        add_chunk(qz * Q + c * CH)
        By[c].wait_recv()
        if c >= 2:
            r = mk(rem_ref, qy * Q + c * CH, 14 + (c - 2), zp)
            r.start()
            Cz.append(r)
        add_chunk(qy * Q + c * CH)

    for c in range(2):
        Cy[c].wait_recv()
        add_chunk(d * Q + c * CH)
        Cz[c].wait_recv()
        add_chunk(d * Q + (2 + c) * CH)

    for r in A + By + Bz + Cy + Cz:
        r.wait_send()


def kernel(x):
    m, n = x.shape
    assert (m, n) == (M, N)

    out, _rem = pl.pallas_call(
        _body,
        out_shape=[
            jax.ShapeDtypeStruct((m, n), x.dtype),
            jax.ShapeDtypeStruct((m, n), x.dtype),
        ],
        in_specs=[pl.BlockSpec(memory_space=pltpu.MemorySpace.HBM)],
        out_specs=[
            pl.BlockSpec(memory_space=pltpu.MemorySpace.HBM),
            pl.BlockSpec(memory_space=pltpu.MemorySpace.HBM),
        ],
        scratch_shapes=[
            pltpu.SemaphoreType.DMA((N_SEMS,)),
            pltpu.SemaphoreType.DMA((N_SEMS,)),
            pltpu.SemaphoreType.DMA((3,)),
            pltpu.VMEM((CH, N), x.dtype),
            pltpu.VMEM((CH, N), x.dtype),
            pltpu.VMEM((CH, N), x.dtype),
        ],
    )(x)
    return out


# device time: 684112 ns/iter; 2.3254x vs baseline; 1.0959x over previous
import jax
import jax.numpy as jnp
from jax import lax
from jax.experimental import pallas as pl
from jax.experimental.pallas import tpu as pltpu


M = 32768
N = 1024
Q = M // 4

ACH = (256, 256, 512, 1024, 2048, 2048, 2048)
AOFF = tuple(sum(ACH[:i]) for i in range(len(ACH)))

DX = ((0, 1536), (1536, 1536))
CY = ((3072, 1024), (4096, 1536))
CZ = ((5632, 512), (6144, 2048))
CY_AFTER = {4: CY[0], 5: CY[1]}
CZ_AFTER = {5: CZ[0], 6: CZ[1]}

N_SEMS = 2 * len(ACH) + len(ACH) + len(DX) + len(CY) + len(CZ)
MAX_ADD_ROWS = 3072


def _body(x_ref, out_ref, rem_ref, recv_sems, send_sems, lsems,
          vx_ref, vr_ref, vo_ref):
    my_x = lax.axis_index("x")
    my_y = lax.axis_index("y")
    my_z = lax.axis_index("z")
    xp = (1 - my_x, my_y, my_z)
    yp = (my_x, 1 - my_y, my_z)
    zp = (my_x, my_y, 1 - my_z)

    q = 2 * my_y + my_z
    qy = 2 * (1 - my_y) + my_z
    qz = 2 * my_y + (1 - my_z)
    d = 2 * (1 - my_y) + (1 - my_z)

    sem_counter = [0]

    def mk(src_base, row, rows, dev):
        i = sem_counter[0]
        sem_counter[0] += 1
        return pltpu.make_async_remote_copy(
            src_ref=src_base.at[pl.ds(row, rows), :],
            dst_ref=rem_ref.at[pl.ds(row, rows), :],
            send_sem=send_sems.at[i],
            recv_sem=recv_sems.at[i],
            device_id=dev,
            device_id_type=pl.DeviceIdType.MESH,
        )

    def add_chunk(row, rows):
        cx = pltpu.make_async_copy(
            x_ref.at[pl.ds(row, rows), :], vx_ref.at[pl.ds(0, rows), :],
            lsems.at[0])
        cr = pltpu.make_async_copy(
            rem_ref.at[pl.ds(row, rows), :], vr_ref.at[pl.ds(0, rows), :],
            lsems.at[1])
        cx.start()
        cr.start()
        cx.wait()
        cr.wait()
        vo_ref[:rows, :] = vx_ref[:rows, :] + vr_ref[:rows, :]
        co = pltpu.make_async_copy(
            vo_ref.at[pl.ds(0, rows), :], out_ref.at[pl.ds(row, rows), :],
            lsems.at[2])
        co.start()
        co.wait()

    A = []
    for c in range(len(ACH)):
        r = mk(x_ref, q * Q + AOFF[c], ACH[c], xp)
        r.start()
        A.append(r)
    Adx = []
    for row, rows in DX:
        r = mk(x_ref, d * Q + row, rows, xp)
        r.start()
        Adx.append(r)

    By, Bz = [], []
    for c in range(len(ACH)):
        A[c].wait_recv()
        ry = mk(rem_ref, q * Q + AOFF[c], ACH[c], yp)
        ry.start()
        By.append(ry)
        rz = mk(rem_ref, q * Q + AOFF[c], ACH[c], zp)
        rz.start()
        Bz.append(rz)
        add_chunk(q * Q + AOFF[c], ACH[c])

    Cy, Cz = [], []
    for c in range(len(ACH)):
        Bz[c].wait_recv()
        if c in CY_AFTER:
            row, rows = CY_AFTER[c]
            r = mk(rem_ref, qz * Q + row, rows, yp)
            r.start()
            Cy.append(r)
        add_chunk(qz * Q + AOFF[c], ACH[c])
        By[c].wait_recv()
        if c in CZ_AFTER:
            row, rows = CZ_AFTER[c]
            r = mk(rem_ref, qy * Q + row, rows, zp)
            r.start()
            Cz.append(r)
        add_chunk(qy * Q + AOFF[c], ACH[c])

    for i, (row, rows) in enumerate(DX):
        Adx[i].wait_recv()
        add_chunk(d * Q + row, rows)
    for i, (row, rows) in enumerate(CY):
        Cy[i].wait_recv()
        add_chunk(d * Q + row, rows)
    for i, (row, rows) in enumerate(CZ):
        Cz[i].wait_recv()
        add_chunk(d * Q + row, rows)

    for r in A + Adx + By + Bz + Cy + Cz:
        r.wait_send()


def kernel(x):
    m, n = x.shape
    assert (m, n) == (M, N)

    out, _rem = pl.pallas_call(
        _body,
        out_shape=[
            jax.ShapeDtypeStruct((m, n), x.dtype),
            jax.ShapeDtypeStruct((m, n), x.dtype),
        ],
        in_specs=[pl.BlockSpec(memory_space=pltpu.MemorySpace.HBM)],
        out_specs=[
            pl.BlockSpec(memory_space=pltpu.MemorySpace.HBM),
            pl.BlockSpec(memory_space=pltpu.MemorySpace.HBM),
        ],
        scratch_shapes=[
            pltpu.SemaphoreType.DMA((N_SEMS,)),
            pltpu.SemaphoreType.DMA((N_SEMS,)),
            pltpu.SemaphoreType.DMA((3,)),
            pltpu.VMEM((MAX_ADD_ROWS, N), x.dtype),
            pltpu.VMEM((MAX_ADD_ROWS, N), x.dtype),
            pltpu.VMEM((MAX_ADD_ROWS, N), x.dtype),
        ],
        compiler_params=pltpu.CompilerParams(
            vmem_limit_bytes=100 * 1024 * 1024,
        ),
    )(x)
    return out
